# Initial kernel scaffold; baseline (speedup 1.0000x reference)
import jax
import jax.numpy as jnp
from jax.experimental import pallas as pl


def kernel(x, edge_index, att, W, b):
    raise NotImplementedError("write your pallas kernel here")


# speedup vs baseline: 1.8711x; 1.8711x over previous
import functools

import jax
import jax.numpy as jnp
from jax import lax
from jax.experimental import pallas as pl
from jax.experimental.pallas import tpu as pltpu
from jax.experimental.pallas import tpu_sc as plsc

B, C, N, K, H = 1, 256, 10000, 16, 1
OUT = 256
NE = N * K
NC, NS = 2, 16
NW = NC * NS
EPW = NE // NW
R = 40
NCHUNK = EPW // R

NB = 200
GRID = N // NB


def _sc_gather_diff(xt, ej, ei):
    mesh = plsc.VectorSubcoreMesh(
        core_axis_name="c", subcore_axis_name="s", num_cores=NC,
        num_subcores=NS)

    @functools.partial(
        pl.kernel,
        out_type=jax.ShapeDtypeStruct((NE, C), jnp.float32),
        mesh=mesh,
        scratch_types=[
            pltpu.VMEM((R,), jnp.int32),
            pltpu.VMEM((R,), jnp.int32),
            pltpu.VMEM((R, C), jnp.float32),
            pltpu.VMEM((R, C), jnp.float32),
            pltpu.SemaphoreType.DMA,
            pltpu.SemaphoreType.DMA,
        ],
    )
    def k(xt_hbm, ej_hbm, ei_hbm, d_hbm, idxj_v, idxi_v, bufj, bufi,
          semj, semi):
        wid = lax.axis_index("s") * NC + lax.axis_index("c")
        base = wid * EPW

        def chunk(t, carry):
            row0 = base + t * R
            pltpu.sync_copy(ej_hbm.at[pl.ds(row0, R)], idxj_v)
            pltpu.sync_copy(ei_hbm.at[pl.ds(row0, R)], idxi_v)
            cj = pltpu.async_copy(xt_hbm.at[idxj_v], bufj, semj)
            ci = pltpu.async_copy(xt_hbm.at[idxi_v], bufi, semi)
            cj.wait()
            ci.wait()

            def rowloop(r, c2):
                for cc in range(C // 16):
                    s = pl.ds(cc * 16, 16)
                    bufj[r, s] = bufj[r, s] - bufi[r, s]
                return c2

            lax.fori_loop(0, R, rowloop, 0)
            pltpu.sync_copy(bufj, d_hbm.at[pl.ds(row0, R)])
            return carry

        lax.fori_loop(0, NCHUNK, chunk, 0)

    return k(xt, ej, ei)


def _tc_body(d_ref, xt_ref, att_ref, wt_ref, b_ref, out_ref):
    d3 = d_ref[...].reshape(NB, K, C)
    act = jnp.where(d3 >= 0, d3, 0.2 * d3)
    logits = jnp.sum(act * att_ref[...][None, :, :], axis=2)
    m = jnp.max(logits, axis=1, keepdims=True)
    e = jnp.exp(logits - m)
    alpha = e / jnp.sum(e, axis=1, keepdims=True)
    xmax = jnp.max(d3 * alpha[:, :, None], axis=1)
    xt = xt_ref[...]
    y = (
        jnp.dot(xt, wt_ref[:C, :], preferred_element_type=jnp.float32)
        + jnp.dot(xmax, wt_ref[C:, :], preferred_element_type=jnp.float32)
        + b_ref[...]
    )
    out_ref[...] = y


def kernel(x, edge_index, att, W, b):
    xt = jnp.transpose(x[0, :, :, 0])
    ej = edge_index[0, 0].reshape(NE)
    ei = edge_index[1, 0].reshape(NE)
    att_v = att.reshape(1, C)
    wt = jnp.transpose(W)
    b2 = b.reshape(1, OUT)

    d = _sc_gather_diff(xt, ej, ei)

    yt = pl.pallas_call(
        _tc_body,
        grid=(GRID,),
        in_specs=[
            pl.BlockSpec((NB * K, C), lambda i: (i, 0)),
            pl.BlockSpec((NB, C), lambda i: (i, 0)),
            pl.BlockSpec((1, C), lambda i: (0, 0)),
            pl.BlockSpec((2 * C, OUT), lambda i: (0, 0)),
            pl.BlockSpec((1, OUT), lambda i: (0, 0)),
        ],
        out_specs=pl.BlockSpec((NB, OUT), lambda i: (i, 0)),
        out_shape=jax.ShapeDtypeStruct((N, OUT), jnp.float32),
    )(d, xt, att_v, wt, b2)

    return jnp.transpose(yt)[None, :, :, None]


# speedup vs baseline: 3.2868x; 1.7567x over previous
import functools

import jax
import jax.numpy as jnp
from jax import lax
from jax.experimental import pallas as pl
from jax.experimental.pallas import tpu as pltpu
from jax.experimental.pallas import tpu_sc as plsc

B, C, N, K, H = 1, 256, 10000, 16, 1
OUT = 256
NE = N * K
NC, NS = 2, 16
NW = NC * NS
EPW = NE // NW
R = 40
NCHUNK = EPW // R

NB = 200
GRID = N // NB


def _sc_gather_diff(xt, ej, ei):
    mesh = plsc.VectorSubcoreMesh(
        core_axis_name="c", subcore_axis_name="s", num_cores=NC,
        num_subcores=NS)

    @functools.partial(
        pl.kernel,
        out_type=jax.ShapeDtypeStruct((NE, C), jnp.float32),
        mesh=mesh,
        scratch_types=[
            pltpu.VMEM((EPW,), jnp.int32),
            pltpu.VMEM((EPW,), jnp.int32),
            [pltpu.VMEM((R, C), jnp.float32)] * 2,
            [pltpu.VMEM((R, C), jnp.float32)] * 2,
            [pltpu.VMEM((R, C), jnp.float32)] * 2,
            [pltpu.SemaphoreType.DMA] * 2,
            [pltpu.SemaphoreType.DMA] * 2,
        ],
    )
    def k(xt_hbm, ej_hbm, ei_hbm, d_hbm, idxj_all, idxi_all, bjs, bis,
          bds, gsems, wsems):
        wid = lax.axis_index("s") * NC + lax.axis_index("c")
        base = wid * EPW
        pltpu.sync_copy(ej_hbm.at[pl.ds(base, EPW)], idxj_all)
        pltpu.sync_copy(ei_hbm.at[pl.ds(base, EPW)], idxi_all)

        def start(t, s):
            off = t * R
            pltpu.async_copy(
                xt_hbm.at[idxj_all.at[pl.ds(off, R)]], bjs[s], gsems[s])
            pltpu.async_copy(
                xt_hbm.at[idxi_all.at[pl.ds(off, R)]], bis[s], gsems[s])

        def finish(t, s):
            bj, bi, bd = bjs[s], bis[s], bds[s]
            pltpu.make_async_copy(
                xt_hbm.at[idxj_all.at[pl.ds(0, R)]], bj, gsems[s]).wait()
            pltpu.make_async_copy(
                xt_hbm.at[idxi_all.at[pl.ds(0, R)]], bi, gsems[s]).wait()

            @pl.when(t >= 2)
            def _():
                pltpu.make_async_copy(
                    bd, d_hbm.at[pl.ds(base, R)], wsems[s]).wait()

            def rowloop(r, c2):
                for cc in range(C // 16):
                    sl = pl.ds(cc * 16, 16)
                    bd[r, sl] = bj[r, sl] - bi[r, sl]
                return c2

            lax.fori_loop(0, R, rowloop, 0)
            pltpu.async_copy(bd, d_hbm.at[pl.ds(base + t * R, R)],
                             wsems[s])

        start(0, 0)
        start(1, 1)

        def body(g, carry):
            for u in (0, 1):
                t = 2 * g + u

                @pl.when(t < NCHUNK)
                def _():
                    finish(t, u)

                @pl.when(t + 2 < NCHUNK)
                def _():
                    start(t + 2, u)
            return carry

        lax.fori_loop(0, (NCHUNK + 1) // 2, body, 0)
        for s in (0, 1):
            pltpu.make_async_copy(
                bds[s], d_hbm.at[pl.ds(base, R)], wsems[s]).wait()

    return k(xt, ej, ei)


def _tc_body(d_ref, xt_ref, att_ref, wt_ref, b_ref, out_ref):
    d3 = d_ref[...].reshape(NB, K, C)
    act = jnp.where(d3 >= 0, d3, 0.2 * d3)
    logits = jnp.sum(act * att_ref[...][None, :, :], axis=2)
    m = jnp.max(logits, axis=1, keepdims=True)
    e = jnp.exp(logits - m)
    alpha = e / jnp.sum(e, axis=1, keepdims=True)
    xmax = jnp.max(d3 * alpha[:, :, None], axis=1)
    xt = xt_ref[...]
    y = (
        jnp.dot(xt, wt_ref[:C, :], preferred_element_type=jnp.float32)
        + jnp.dot(xmax, wt_ref[C:, :], preferred_element_type=jnp.float32)
        + b_ref[...]
    )
    out_ref[...] = y


def kernel(x, edge_index, att, W, b):
    xt = jnp.transpose(x[0, :, :, 0])
    ej = edge_index[0, 0].reshape(NE)
    ei = edge_index[1, 0].reshape(NE)
    att_v = att.reshape(1, C)
    wt = jnp.transpose(W)
    b2 = b.reshape(1, OUT)

    d = _sc_gather_diff(xt, ej, ei)

    yt = pl.pallas_call(
        _tc_body,
        grid=(GRID,),
        in_specs=[
            pl.BlockSpec((NB * K, C), lambda i: (i, 0)),
            pl.BlockSpec((NB, C), lambda i: (i, 0)),
            pl.BlockSpec((1, C), lambda i: (0, 0)),
            pl.BlockSpec((2 * C, OUT), lambda i: (0, 0)),
            pl.BlockSpec((1, OUT), lambda i: (0, 0)),
        ],
        out_specs=pl.BlockSpec((NB, OUT), lambda i: (i, 0)),
        out_shape=jax.ShapeDtypeStruct((N, OUT), jnp.float32),
    )(d, xt, att_v, wt, b2)

    return jnp.transpose(yt)[None, :, :, None]


# speedup vs baseline: 3.4059x; 1.0362x over previous
import functools

import jax
import jax.numpy as jnp
from jax import lax
from jax.experimental import pallas as pl
from jax.experimental.pallas import tpu as pltpu
from jax.experimental.pallas import tpu_sc as plsc

B, C, N, K, H = 1, 256, 10000, 16, 1
OUT = 256
NE = N * K
NC, NS = 2, 16
NW = NC * NS

P = 5
N_P = N // P
NE_P = NE // P
EPW = NE_P // NW
R = 40
NCHUNK = EPW // R

NB = 200
GRID_P = N_P // NB


def _make_sc_gather_diff(part):
    mesh = plsc.VectorSubcoreMesh(
        core_axis_name="c", subcore_axis_name="s", num_cores=NC,
        num_subcores=NS)

    @functools.partial(
        pl.kernel,
        out_type=jax.ShapeDtypeStruct((NE_P, C), jnp.float32),
        mesh=mesh,
        scratch_types=[
            pltpu.VMEM((EPW,), jnp.int32),
            pltpu.VMEM((EPW,), jnp.int32),
            [pltpu.VMEM((R, C), jnp.float32)] * 2,
            [pltpu.VMEM((R, C), jnp.float32)] * 2,
            [pltpu.VMEM((R, C), jnp.float32)] * 2,
            [pltpu.SemaphoreType.DMA] * 2,
            [pltpu.SemaphoreType.DMA] * 2,
        ],
    )
    def k(xt_hbm, ej_hbm, ei_hbm, d_hbm, idxj_all, idxi_all, bjs, bis,
          bds, gsems, wsems):
        wid = lax.axis_index("s") * NC + lax.axis_index("c")
        src_base = part * NE_P + wid * EPW
        dst_base = wid * EPW
        pltpu.sync_copy(ej_hbm.at[pl.ds(src_base, EPW)], idxj_all)
        pltpu.sync_copy(ei_hbm.at[pl.ds(src_base, EPW)], idxi_all)

        def start(t, s):
            off = t * R
            pltpu.async_copy(
                xt_hbm.at[idxj_all.at[pl.ds(off, R)]], bjs[s], gsems[s])
            pltpu.async_copy(
                xt_hbm.at[idxi_all.at[pl.ds(off, R)]], bis[s], gsems[s])

        def finish(t, s):
            bj, bi, bd = bjs[s], bis[s], bds[s]
            pltpu.make_async_copy(
                xt_hbm.at[idxj_all.at[pl.ds(0, R)]], bj, gsems[s]).wait()
            pltpu.make_async_copy(
                xt_hbm.at[idxi_all.at[pl.ds(0, R)]], bi, gsems[s]).wait()

            @pl.when(t >= 2)
            def _():
                pltpu.make_async_copy(
                    bd, d_hbm.at[pl.ds(dst_base, R)], wsems[s]).wait()

            def rowloop(r, c2):
                for cc in range(C // 16):
                    sl = pl.ds(cc * 16, 16)
                    bd[r, sl] = bj[r, sl] - bi[r, sl]
                return c2

            lax.fori_loop(0, R, rowloop, 0)
            pltpu.async_copy(bd, d_hbm.at[pl.ds(dst_base + t * R, R)],
                             wsems[s])

        start(0, 0)
        start(1, 1)

        def body(g, carry):
            for u in (0, 1):
                t = 2 * g + u

                @pl.when(t < NCHUNK)
                def _():
                    finish(t, u)

                @pl.when(t + 2 < NCHUNK)
                def _():
                    start(t + 2, u)
            return carry

        lax.fori_loop(0, (NCHUNK + 1) // 2, body, 0)
        for s in (0, 1):
            pltpu.make_async_copy(
                bds[s], d_hbm.at[pl.ds(dst_base, R)], wsems[s]).wait()

    return k


def _tc_body(d_ref, xt_ref, att_ref, wt_ref, b_ref, out_ref):
    d3 = d_ref[...].reshape(NB, K, C)
    act = jnp.where(d3 >= 0, d3, 0.2 * d3)
    logits = jnp.sum(act * att_ref[...][None, :, :], axis=2)
    m = jnp.max(logits, axis=1, keepdims=True)
    e = jnp.exp(logits - m)
    alpha = e / jnp.sum(e, axis=1, keepdims=True)
    xmax = jnp.max(d3 * alpha[:, :, None], axis=1)
    xt = xt_ref[...]
    y = (
        jnp.dot(xt, wt_ref[:C, :], preferred_element_type=jnp.float32)
        + jnp.dot(xmax, wt_ref[C:, :], preferred_element_type=jnp.float32)
        + b_ref[...]
    )
    out_ref[...] = y


def _tc_part(part, d, xt, att_v, wt, b2):
    return pl.pallas_call(
        _tc_body,
        grid=(GRID_P,),
        in_specs=[
            pl.BlockSpec((NB * K, C), lambda i: (i, 0)),
            pl.BlockSpec((NB, C), lambda i, p=part: (i + p * GRID_P, 0)),
            pl.BlockSpec((1, C), lambda i: (0, 0)),
            pl.BlockSpec((2 * C, OUT), lambda i: (0, 0)),
            pl.BlockSpec((1, OUT), lambda i: (0, 0)),
        ],
        out_specs=pl.BlockSpec((NB, OUT), lambda i: (i, 0)),
        out_shape=jax.ShapeDtypeStruct((N_P, OUT), jnp.float32),
    )(d, xt, att_v, wt, b2)


def kernel(x, edge_index, att, W, b):
    xt = jnp.transpose(x[0, :, :, 0])
    ej = edge_index[0, 0].reshape(NE)
    ei = edge_index[1, 0].reshape(NE)
    att_v = att.reshape(1, C)
    wt = jnp.transpose(W)
    b2 = b.reshape(1, OUT)

    ds = [_make_sc_gather_diff(p)(xt, ej, ei) for p in range(P)]
    yts = [_tc_part(p, ds[p], xt, att_v, wt, b2) for p in range(P)]
    yt = jnp.concatenate(yts, axis=0)
    return jnp.transpose(yt)[None, :, :, None]
